# Initial kernel scaffold; baseline (speedup 1.0000x reference)
import functools
import jax
import jax.numpy as jnp
from jax import lax
from jax.experimental import pallas as pl
from jax.experimental.pallas import tpu as pltpu
from jax.experimental.pallas import tpu_sc as plsc

N = 100000
E = 1600000
IN_DIM = 16
D = 50
NUM_TASK = 112

NTILES = 32
EDGES_PER_TILE = E // NTILES
CHUNK = 125
JROWS = 16
SUPER = CHUNK * JROWS
SUPERS_PER_TILE = EDGES_PER_TILE // SUPER
NSUP = E // SUPER
ROWS_PER_TILE = N // 16
DP = 17

_mesh = plsc.VectorSubcoreMesh(core_axis_name="c", subcore_axis_name="s")


def _tile_ids():
    c = lax.axis_index("c")
    s = lax.axis_index("s")
    return c, s


def _sc_feats_body(ef_hbm, dst3_hbm, z16_hbm, z1_hbm, ones_hbm,
                   out_feat, out_deg, acc, dacc, dbuf, fbuf, obuf, sem):
    c, s = _tile_ids()
    row0 = s * ROWS_PER_TILE
    pltpu.sync_copy(z16_hbm.at[pl.ds(row0, ROWS_PER_TILE), :],
                    acc.at[pl.ds(row0, ROWS_PER_TILE), :])
    pltpu.sync_copy(z1_hbm.at[pl.ds(row0, ROWS_PER_TILE), :],
                    dacc.at[pl.ds(row0, ROWS_PER_TILE), :])
    pltpu.sync_copy(ones_hbm, obuf)
    plsc.subcore_barrier()

    base_super = (c * 16 + s) * SUPERS_PER_TILE

    def body(k2, carry):
        g = base_super + k2
        pltpu.sync_copy(dst3_hbm.at[g], dbuf)
        for j in range(JROWS):
            e0 = (g * JROWS + j) * CHUNK
            pltpu.sync_copy(ef_hbm.at[pl.ds(e0, CHUNK), :], fbuf)
            pltpu.sync_copy(fbuf, acc.at[dbuf.at[j]], add=True)
            pltpu.sync_copy(obuf, dacc.at[dbuf.at[j]], add=True)
        return carry

    lax.fori_loop(0, SUPERS_PER_TILE, body, 0)
    plsc.subcore_barrier()
    pltpu.sync_copy(acc.at[pl.ds(row0, ROWS_PER_TILE), :],
                    out_feat.at[c, pl.ds(row0, ROWS_PER_TILE), :])
    pltpu.sync_copy(dacc.at[pl.ds(row0, ROWS_PER_TILE), :],
                    out_deg.at[c, pl.ds(row0, ROWS_PER_TILE), :])


_sc_feats = functools.partial(
    pl.kernel,
    mesh=_mesh,
    out_type=[
        jax.ShapeDtypeStruct((2, N, IN_DIM), jnp.float32),
        jax.ShapeDtypeStruct((2, N, 1), jnp.float32),
    ],
    scratch_types=[
        pltpu.VMEM_SHARED((N, IN_DIM), jnp.float32),
        pltpu.VMEM_SHARED((N, 1), jnp.float32),
        pltpu.VMEM((JROWS, CHUNK), jnp.int32),
        pltpu.VMEM((CHUNK, IN_DIM), jnp.float32),
        pltpu.VMEM((CHUNK, 1), jnp.float32),
        pltpu.SemaphoreType.DMA,
    ],
)(_sc_feats_body)


def _sc_spmm_body(ha_hbm, hb_hbm, hc_hbm, src3_hbm, dst3_hbm, z17_hbm,
                  outA, outB, outC, acc, sbuf, dbuf, vbuf, sem):
    c, s = _tile_ids()
    row0 = s * ROWS_PER_TILE
    base_super = (c * 16 + s) * SUPERS_PER_TILE

    for tab, out in ((ha_hbm, outA), (hb_hbm, outB), (hc_hbm, outC)):
        pltpu.sync_copy(z17_hbm.at[pl.ds(row0, ROWS_PER_TILE), :],
                        acc.at[pl.ds(row0, ROWS_PER_TILE), :])
        plsc.subcore_barrier()

        def body(k2, carry):
            g = base_super + k2
            pltpu.sync_copy(src3_hbm.at[g], sbuf)
            pltpu.sync_copy(dst3_hbm.at[g], dbuf)
            for j in range(JROWS):
                pltpu.async_copy(tab.at[sbuf.at[j]], vbuf, sem).wait()
                pltpu.sync_copy(vbuf, acc.at[dbuf.at[j]], add=True)
            return carry

        lax.fori_loop(0, SUPERS_PER_TILE, body, 0)
        plsc.subcore_barrier()
        pltpu.sync_copy(acc.at[pl.ds(row0, ROWS_PER_TILE), :],
                        out.at[c, pl.ds(row0, ROWS_PER_TILE), :])
        plsc.subcore_barrier()


_sc_spmm = functools.partial(
    pl.kernel,
    mesh=_mesh,
    out_type=[
        jax.ShapeDtypeStruct((2, N, DP), jnp.float32),
        jax.ShapeDtypeStruct((2, N, DP), jnp.float32),
        jax.ShapeDtypeStruct((2, N, DP), jnp.float32),
    ],
    scratch_types=[
        pltpu.VMEM_SHARED((N, DP), jnp.float32),
        pltpu.VMEM((JROWS, CHUNK), jnp.int32),
        pltpu.VMEM((JROWS, CHUNK), jnp.int32),
        pltpu.VMEM((CHUNK, DP), jnp.float32),
        pltpu.SemaphoreType.DMA,
    ],
)(_sc_spmm_body)


BROWS = 2000
GRID = N // BROWS
FN = jnp.float32


def _row_spec(width):
    return pl.BlockSpec((BROWS, width), lambda i: (i, 0))


def _part_spec(width):
    return pl.BlockSpec((2, BROWS, width), lambda i: (0, i, 0))


def _full_spec(a, b):
    return pl.BlockSpec((a, b), lambda i: (0, 0))


def _p0_body(fp, dp, nemb, We, be, eps0, W1, b1, se_out, x1_out, st_out):
    i = pl.program_id(0)
    f = fp[0] + fp[1]
    deg = dp[0, :, 0] + dp[1, :, 0]
    Se = jnp.dot(f, We[...], preferred_element_type=FN) + deg[:, None] * be[...]
    se_out[...] = Se
    maxdeg = jnp.maximum(deg, 1.0)[:, None]
    r = nemb[...]
    h = (1.0 + eps0[0, 0]) * r + (deg[:, None] * r + Se) / maxdeg
    x = jnp.dot(h, W1[...], preferred_element_type=FN) + b1[...]
    x1_out[...] = x
    st = jnp.concatenate([jnp.sum(x, axis=0, keepdims=True),
                          jnp.sum(x * x, axis=0, keepdims=True)], axis=0)

    @pl.when(i == 0)
    def _():
        st_out[...] = jnp.zeros_like(st_out)

    st_out[...] += st


def _p0(featp, degp, nemb, We, be, eps0, W1, b1):
    return pl.pallas_call(
        _p0_body,
        grid=(GRID,),
        in_specs=[
            _part_spec(IN_DIM), _part_spec(1), _full_spec(1, D),
            _full_spec(IN_DIM, D), _full_spec(1, D), _full_spec(1, 1),
            _full_spec(D, 2 * D), _full_spec(1, 2 * D),
        ],
        out_specs=[
            _row_spec(D), _row_spec(2 * D),
            pl.BlockSpec((2, 2 * D), lambda i: (0, 0)),
        ],
        out_shape=[
            jax.ShapeDtypeStruct((N, D), FN),
            jax.ShapeDtypeStruct((N, 2 * D), FN),
            jax.ShapeDtypeStruct((2, 2 * D), FN),
        ],
    )(featp, degp, nemb, We, be, eps0, W1, b1)


def _bn_from_stats(x, st, g, bt):
    mean = st[0:1, :] * (1.0 / N)
    var = st[1:2, :] * (1.0 / N) - mean * mean
    inv = lax.rsqrt(var + 1e-5)
    return (x - mean) * inv * g + bt


def _p1_body(x_in, st_in, g, bt, W2, b2, y_out, st_out):
    i = pl.program_id(0)
    xn = _bn_from_stats(x_in[...], st_in[...], g[...], bt[...])
    xn = jnp.maximum(xn, 0.0)
    y = jnp.dot(xn, W2[...], preferred_element_type=FN) + b2[...]
    y_out[...] = y
    st = jnp.concatenate([jnp.sum(y, axis=0, keepdims=True),
                          jnp.sum(y * y, axis=0, keepdims=True)], axis=0)

    @pl.when(i == 0)
    def _():
        st_out[...] = jnp.zeros_like(st_out)

    st_out[...] += st


def _p1(x, st, g, bt, W2, b2, din, dout):
    return pl.pallas_call(
        _p1_body,
        grid=(GRID,),
        in_specs=[
            _row_spec(din), pl.BlockSpec((2, din), lambda i: (0, 0)),
            _full_spec(1, din), _full_spec(1, din),
            _full_spec(din, dout), _full_spec(1, dout),
        ],
        out_specs=[_row_spec(dout), pl.BlockSpec((2, dout), lambda i: (0, 0))],
        out_shape=[
            jax.ShapeDtypeStruct((N, dout), FN),
            jax.ShapeDtypeStruct((2, dout), FN),
        ],
    )(x, st, g, bt, W2, b2)


def _p2_body(x_in, st_in, g, bt, hv_out, ha_out, hb_out, hc_out):
    h = _bn_from_stats(x_in[...], st_in[...], g[...], bt[...])
    h = jnp.maximum(h, 0.0)
    hv_out[...] = h
    ha_out[...] = h[:, 0:DP]
    hb_out[...] = h[:, DP:2 * DP]
    hc_out[...] = jnp.concatenate(
        [h[:, 2 * DP:D], jnp.zeros((BROWS, 3 * DP - D), FN)], axis=1)


def _p2(x, st, g, bt):
    return pl.pallas_call(
        _p2_body,
        grid=(GRID,),
        in_specs=[
            _row_spec(D), pl.BlockSpec((2, D), lambda i: (0, 0)),
            _full_spec(1, D), _full_spec(1, D),
        ],
        out_specs=[_row_spec(D), _row_spec(DP), _row_spec(DP), _row_spec(DP)],
        out_shape=[
            jax.ShapeDtypeStruct((N, D), FN),
            jax.ShapeDtypeStruct((N, DP), FN),
            jax.ShapeDtypeStruct((N, DP), FN),
            jax.ShapeDtypeStruct((N, DP), FN),
        ],
    )(x, st, g, bt)


def _p3_body(shA, shB, shC, se_in, hv_in, dp, eps1, W1, b1, x_out, st_out):
    i = pl.program_id(0)
    deg = dp[0, :, 0] + dp[1, :, 0]
    maxdeg = jnp.maximum(deg, 1.0)[:, None]
    Sh = jnp.concatenate([shA[0] + shA[1], shB[0] + shB[1],
                          (shC[0] + shC[1])[:, 0:D - 2 * DP]], axis=1)
    hv = hv_in[...]
    h = (1.0 + eps1[0, 0]) * hv + (Sh + se_in[...]) / maxdeg
    x = jnp.dot(h, W1[...], preferred_element_type=FN) + b1[...]
    x_out[...] = x
    st = jnp.concatenate([jnp.sum(x, axis=0, keepdims=True),
                          jnp.sum(x * x, axis=0, keepdims=True)], axis=0)

    @pl.when(i == 0)
    def _():
        st_out[...] = jnp.zeros_like(st_out)

    st_out[...] += st


def _p3(ShA, ShB, ShC, Se, hv1, degp, eps1, W1, b1):
    return pl.pallas_call(
        _p3_body,
        grid=(GRID,),
        in_specs=[
            _part_spec(DP), _part_spec(DP), _part_spec(DP),
            _row_spec(D), _row_spec(D), _part_spec(1),
            _full_spec(1, 1), _full_spec(D, 2 * D), _full_spec(1, 2 * D),
        ],
        out_specs=[_row_spec(2 * D), pl.BlockSpec((2, 2 * D), lambda i: (0, 0))],
        out_shape=[
            jax.ShapeDtypeStruct((N, 2 * D), FN),
            jax.ShapeDtypeStruct((2, 2 * D), FN),
        ],
    )(ShA, ShB, ShC, Se, hv1, degp, eps1, W1, b1)


def _p5_body(x_in, st_in, g, bt, Wp, bp, o_out):
    h = _bn_from_stats(x_in[...], st_in[...], g[...], bt[...])
    h = jnp.maximum(h, 0.0)
    o_out[...] = jnp.dot(h, Wp[...], preferred_element_type=FN) + bp[...]


def _p5(x, st, g, bt, Wp, bp):
    return pl.pallas_call(
        _p5_body,
        grid=(GRID,),
        in_specs=[
            _row_spec(D), pl.BlockSpec((2, D), lambda i: (0, 0)),
            _full_spec(1, D), _full_spec(1, D),
            _full_spec(D, NUM_TASK), _full_spec(1, NUM_TASK),
        ],
        out_specs=[_row_spec(NUM_TASK)],
        out_shape=[jax.ShapeDtypeStruct((N, NUM_TASK), FN)],
    )(x, st, g, bt, Wp, bp)


def kernel(edge_index, edge_feats, node_emb, We, be, eps0, W1_0, b1_0, g1_0,
           bt1_0, W2_0, b2_0, g2_0, bt2_0, eps1, W1_1, b1_1, g1_1, bt1_1,
           W2_1, b2_1, g2_1, bt2_1, Wp, bp):
    src3 = edge_index[0].reshape(NSUP, JROWS, CHUNK)
    dst3 = edge_index[1].reshape(NSUP, JROWS, CHUNK)
    z16 = jnp.zeros((N, IN_DIM), FN)
    z1 = jnp.zeros((N, 1), FN)
    z17 = jnp.zeros((N, DP), FN)
    ones = jnp.ones((CHUNK, 1), FN)

    r2 = lambda v: v.reshape(1, -1)

    featp, degp = _sc_feats(edge_feats, dst3, z16, z1, ones)
    Se, X1, st1 = _p0(featp, degp, node_emb, We, r2(be), r2(eps0),
                      W1_0, r2(b1_0))
    X2, st2 = _p1(X1, st1, r2(g1_0), r2(bt1_0), W2_0, r2(b2_0), 2 * D, D)
    hv1, ha, hb, hc = _p2(X2, st2, r2(g2_0), r2(bt2_0))
    ShA, ShB, ShC = _sc_spmm(ha, hb, hc, src3, dst3, z17)
    X3, st3 = _p3(ShA, ShB, ShC, Se, hv1, degp, r2(eps1), W1_1, r2(b1_1))
    X4, st4 = _p1(X3, st3, r2(g1_1), r2(bt1_1), W2_1, r2(b2_1), 2 * D, D)
    return _p5(X4, st4, r2(g2_1), r2(bt2_1), Wp, bp)


# speedup vs baseline: 2.1892x; 2.1892x over previous
import functools
import jax
import jax.numpy as jnp
from jax import lax
from jax.experimental import pallas as pl
from jax.experimental.pallas import tpu as pltpu
from jax.experimental.pallas import tpu_sc as plsc

N = 100000
E = 1600000
IN_DIM = 16
D = 50
NUM_TASK = 112
FN = jnp.float32

CHUNK = 128
GCHUNKS = 8
EGROUP = CHUNK * GCHUNKS
GROUPS_PER_TILE = 49
NGROUPS = 32 * GROUPS_PER_TILE
EPAD = NGROUPS * EGROUP
ROWS_PER_TILE = 6256
NPAD = 16 * ROWS_PER_TILE
CW = 16
NCH = 4

_mesh = plsc.VectorSubcoreMesh(core_axis_name="c", subcore_axis_name="s")
_sc_params = pltpu.CompilerParams(use_tc_tiling_on_sc=False)

_ACC_T = jax.ShapeDtypeStruct((2, NPAD, CW), jnp.float32)


def _sc_scatter_body(h0, h1, h2, h3, dst2, z16, o0, o1, o2, o3,
                     acc, ibuf, fbuf, sem):
    c, s = lax.axis_index("c"), lax.axis_index("s")
    row0 = s * ROWS_PER_TILE
    g0 = (c * 16 + s) * GROUPS_PER_TILE

    for tab, out in ((h0, o0), (h1, o1), (h2, o2), (h3, o3)):
        pltpu.sync_copy(z16.at[pl.ds(row0, ROWS_PER_TILE), :],
                        acc.at[pl.ds(row0, ROWS_PER_TILE), :])
        plsc.subcore_barrier()

        def body(k2, carry):
            g = g0 + k2
            pltpu.sync_copy(dst2.at[pl.ds(g * GCHUNKS, GCHUNKS), :], ibuf)
            pltpu.sync_copy(tab.at[pl.ds(g * EGROUP, EGROUP), :], fbuf)
            for j in range(GCHUNKS):
                pltpu.sync_copy(fbuf.at[pl.ds(j * CHUNK, CHUNK), :],
                                acc.at[ibuf.at[j]], add=True)
            return carry

        lax.fori_loop(0, GROUPS_PER_TILE, body, 0)
        plsc.subcore_barrier()
        pltpu.sync_copy(acc.at[pl.ds(row0, ROWS_PER_TILE), :],
                        out.at[c, pl.ds(row0, ROWS_PER_TILE), :])
        plsc.subcore_barrier()


_sc_scatter = functools.partial(
    pl.kernel,
    mesh=_mesh,
    out_type=[_ACC_T, _ACC_T, _ACC_T, _ACC_T],
    scratch_types=[
        pltpu.VMEM_SHARED((NPAD, CW), jnp.float32),
        pltpu.VMEM((GCHUNKS, CHUNK), jnp.int32),
        pltpu.VMEM((EGROUP, CW), jnp.float32),
        pltpu.SemaphoreType.DMA,
    ],
    compiler_params=_sc_params,
)(_sc_scatter_body)


def _sc_spmm_body(t0, t1, t2, t3, src2, dst2, z16, o0, o1, o2, o3,
                  acc, sibuf, dibuf, vbuf, sem):
    c, s = lax.axis_index("c"), lax.axis_index("s")
    row0 = s * ROWS_PER_TILE
    g0 = (c * 16 + s) * GROUPS_PER_TILE

    for tab, out in ((t0, o0), (t1, o1), (t2, o2), (t3, o3)):
        pltpu.sync_copy(z16.at[pl.ds(row0, ROWS_PER_TILE), :],
                        acc.at[pl.ds(row0, ROWS_PER_TILE), :])
        plsc.subcore_barrier()

        def body(k2, carry):
            g = g0 + k2
            pltpu.sync_copy(src2.at[pl.ds(g * GCHUNKS, GCHUNKS), :], sibuf)
            pltpu.sync_copy(dst2.at[pl.ds(g * GCHUNKS, GCHUNKS), :], dibuf)
            for j in range(GCHUNKS):
                pltpu.async_copy(tab.at[sibuf.at[j]], vbuf, sem).wait()
                pltpu.sync_copy(vbuf, acc.at[dibuf.at[j]], add=True)
            return carry

        lax.fori_loop(0, GROUPS_PER_TILE, body, 0)
        plsc.subcore_barrier()
        pltpu.sync_copy(acc.at[pl.ds(row0, ROWS_PER_TILE), :],
                        out.at[c, pl.ds(row0, ROWS_PER_TILE), :])
        plsc.subcore_barrier()


_sc_spmm = functools.partial(
    pl.kernel,
    mesh=_mesh,
    out_type=[_ACC_T, _ACC_T, _ACC_T, _ACC_T],
    scratch_types=[
        pltpu.VMEM_SHARED((NPAD, CW), jnp.float32),
        pltpu.VMEM((GCHUNKS, CHUNK), jnp.int32),
        pltpu.VMEM((GCHUNKS, CHUNK), jnp.int32),
        pltpu.VMEM((CHUNK, CW), jnp.float32),
        pltpu.SemaphoreType.DMA,
    ],
    compiler_params=_sc_params,
)(_sc_spmm_body)


BROWS = 2000
GRID = N // BROWS
EROWS = 2048
EGRID = EPAD // EROWS


def _row_spec(width, rows=BROWS):
    return pl.BlockSpec((rows, width), lambda i: (i, 0))


def _part_spec(width):
    return pl.BlockSpec((2, BROWS, width), lambda i: (0, i, 0))


def _full_spec(a, b):
    return pl.BlockSpec((a, b), lambda i: (0, 0))


def _he_body(ef, We, be, c0, c1, c2, c3):
    x = jnp.dot(ef[...], We[...], preferred_element_type=FN) + be[...]
    c0[...] = x[:, 0:16]
    c1[...] = x[:, 16:32]
    c2[...] = x[:, 32:48]
    c3[...] = jnp.concatenate(
        [x[:, 48:50], jnp.ones((EROWS, 1), FN), jnp.zeros((EROWS, 13), FN)],
        axis=1)


def _he(efp, We, be):
    return pl.pallas_call(
        _he_body,
        grid=(EGRID,),
        in_specs=[_row_spec(IN_DIM, EROWS), _full_spec(IN_DIM, D),
                  _full_spec(1, D)],
        out_specs=[_row_spec(CW, EROWS)] * NCH,
        out_shape=[jax.ShapeDtypeStruct((EPAD, CW), FN)] * NCH,
    )(efp, We, be)


def _p0_body(c0p, c1p, c2p, c3p, nemb, eps0, W1, b1,
             se_out, x1_out, st_out, deg_out):
    i = pl.program_id(0)
    s3 = c3p[0] + c3p[1]
    deg = s3[:, 2]
    deg_out[...] = deg[:, None]
    Se = jnp.concatenate(
        [c0p[0] + c0p[1], c1p[0] + c1p[1], c2p[0] + c2p[1], s3[:, 0:2]],
        axis=1)
    se_out[...] = Se
    maxdeg = jnp.maximum(deg, 1.0)[:, None]
    r = nemb[...]
    h = (1.0 + eps0[0, 0]) * r + (deg[:, None] * r + Se) / maxdeg
    x = jnp.dot(h, W1[...], preferred_element_type=FN) + b1[...]
    x1_out[...] = x
    st = jnp.concatenate([jnp.sum(x, axis=0, keepdims=True),
                          jnp.sum(x * x, axis=0, keepdims=True)], axis=0)

    @pl.when(i == 0)
    def _():
        st_out[...] = jnp.zeros_like(st_out)

    st_out[...] += st


def _p0(sep, nemb, eps0, W1, b1):
    return pl.pallas_call(
        _p0_body,
        grid=(GRID,),
        in_specs=[
            _part_spec(CW), _part_spec(CW), _part_spec(CW), _part_spec(CW),
            _full_spec(1, D), _full_spec(1, 1),
            _full_spec(D, 2 * D), _full_spec(1, 2 * D),
        ],
        out_specs=[
            _row_spec(D), _row_spec(2 * D),
            pl.BlockSpec((2, 2 * D), lambda i: (0, 0)), _row_spec(1),
        ],
        out_shape=[
            jax.ShapeDtypeStruct((N, D), FN),
            jax.ShapeDtypeStruct((N, 2 * D), FN),
            jax.ShapeDtypeStruct((2, 2 * D), FN),
            jax.ShapeDtypeStruct((N, 1), FN),
        ],
    )(sep[0], sep[1], sep[2], sep[3], nemb, eps0, W1, b1)


def _bn_from_stats(x, st, g, bt):
    mean = st[0:1, :] * (1.0 / N)
    var = st[1:2, :] * (1.0 / N) - mean * mean
    inv = lax.rsqrt(var + 1e-5)
    return (x - mean) * inv * g + bt


def _p1_body(x_in, st_in, g, bt, W2, b2, y_out, st_out):
    i = pl.program_id(0)
    xn = _bn_from_stats(x_in[...], st_in[...], g[...], bt[...])
    xn = jnp.maximum(xn, 0.0)
    y = jnp.dot(xn, W2[...], preferred_element_type=FN) + b2[...]
    y_out[...] = y
    st = jnp.concatenate([jnp.sum(y, axis=0, keepdims=True),
                          jnp.sum(y * y, axis=0, keepdims=True)], axis=0)

    @pl.when(i == 0)
    def _():
        st_out[...] = jnp.zeros_like(st_out)

    st_out[...] += st


def _p1(x, st, g, bt, W2, b2, din, dout):
    return pl.pallas_call(
        _p1_body,
        grid=(GRID,),
        in_specs=[
            _row_spec(din), pl.BlockSpec((2, din), lambda i: (0, 0)),
            _full_spec(1, din), _full_spec(1, din),
            _full_spec(din, dout), _full_spec(1, dout),
        ],
        out_specs=[_row_spec(dout), pl.BlockSpec((2, dout), lambda i: (0, 0))],
        out_shape=[
            jax.ShapeDtypeStruct((N, dout), FN),
            jax.ShapeDtypeStruct((2, dout), FN),
        ],
    )(x, st, g, bt, W2, b2)


def _p2_body(x_in, st_in, g, bt, hv_out, t0, t1, t2, t3):
    h = _bn_from_stats(x_in[...], st_in[...], g[...], bt[...])
    h = jnp.maximum(h, 0.0)
    hv_out[...] = h
    t0[...] = h[:, 0:16]
    t1[...] = h[:, 16:32]
    t2[...] = h[:, 32:48]
    t3[...] = jnp.concatenate([h[:, 48:50], jnp.zeros((BROWS, 14), FN)],
                              axis=1)


def _p2(x, st, g, bt):
    return pl.pallas_call(
        _p2_body,
        grid=(GRID,),
        in_specs=[
            _row_spec(D), pl.BlockSpec((2, D), lambda i: (0, 0)),
            _full_spec(1, D), _full_spec(1, D),
        ],
        out_specs=[_row_spec(D)] + [_row_spec(CW)] * NCH,
        out_shape=[jax.ShapeDtypeStruct((N, D), FN)]
        + [jax.ShapeDtypeStruct((N, CW), FN)] * NCH,
    )(x, st, g, bt)


def _p3_body(q0, q1, q2, q3, se_in, hv_in, dg, eps1, W1, b1, x_out, st_out):
    i = pl.program_id(0)
    deg = dg[:, 0]
    maxdeg = jnp.maximum(deg, 1.0)[:, None]
    Sh = jnp.concatenate(
        [q0[0] + q0[1], q1[0] + q1[1], q2[0] + q2[1],
         (q3[0] + q3[1])[:, 0:2]], axis=1)
    h = (1.0 + eps1[0, 0]) * hv_in[...] + (Sh + se_in[...]) / maxdeg
    x = jnp.dot(h, W1[...], preferred_element_type=FN) + b1[...]
    x_out[...] = x
    st = jnp.concatenate([jnp.sum(x, axis=0, keepdims=True),
                          jnp.sum(x * x, axis=0, keepdims=True)], axis=0)

    @pl.when(i == 0)
    def _():
        st_out[...] = jnp.zeros_like(st_out)

    st_out[...] += st


def _p3(shp, Se, hv1, deg, eps1, W1, b1):
    return pl.pallas_call(
        _p3_body,
        grid=(GRID,),
        in_specs=[
            _part_spec(CW), _part_spec(CW), _part_spec(CW), _part_spec(CW),
            _row_spec(D), _row_spec(D), _row_spec(1),
            _full_spec(1, 1), _full_spec(D, 2 * D), _full_spec(1, 2 * D),
        ],
        out_specs=[_row_spec(2 * D), pl.BlockSpec((2, 2 * D), lambda i: (0, 0))],
        out_shape=[
            jax.ShapeDtypeStruct((N, 2 * D), FN),
            jax.ShapeDtypeStruct((2, 2 * D), FN),
        ],
    )(shp[0], shp[1], shp[2], shp[3], Se, hv1, deg, eps1, W1, b1)


def _p5_body(x_in, st_in, g, bt, Wp, bp, o_out):
    h = _bn_from_stats(x_in[...], st_in[...], g[...], bt[...])
    h = jnp.maximum(h, 0.0)
    o_out[...] = jnp.dot(h, Wp[...], preferred_element_type=FN) + bp[...]


def _p5(x, st, g, bt, Wp, bp):
    return pl.pallas_call(
        _p5_body,
        grid=(GRID,),
        in_specs=[
            _row_spec(D), pl.BlockSpec((2, D), lambda i: (0, 0)),
            _full_spec(1, D), _full_spec(1, D),
            _full_spec(D, NUM_TASK), _full_spec(1, NUM_TASK),
        ],
        out_specs=[_row_spec(NUM_TASK)],
        out_shape=[jax.ShapeDtypeStruct((N, NUM_TASK), FN)],
    )(x, st, g, bt, Wp, bp)[0]


def kernel(edge_index, edge_feats, node_emb, We, be, eps0, W1_0, b1_0, g1_0,
           bt1_0, W2_0, b2_0, g2_0, bt2_0, eps1, W1_1, b1_1, g1_1, bt1_1,
           W2_1, b2_1, g2_1, bt2_1, Wp, bp):
    npad_e = EPAD - E
    src2 = jnp.concatenate(
        [edge_index[0], jnp.zeros((npad_e,), jnp.int32)]).reshape(-1, CHUNK)
    dst2 = jnp.concatenate(
        [edge_index[1], jnp.full((npad_e,), N, jnp.int32)]).reshape(-1, CHUNK)
    efp = jnp.concatenate([edge_feats, jnp.zeros((npad_e, IN_DIM), FN)], 0)
    z16 = jnp.zeros((NPAD, CW), FN)

    r2 = lambda v: v.reshape(1, -1)

    hec = _he(efp, We, r2(be))
    sep = _sc_scatter(hec[0], hec[1], hec[2], hec[3], dst2, z16)
    Se, X1, st1, deg = _p0(sep, node_emb, r2(eps0), W1_0, r2(b1_0))
    X2, st2 = _p1(X1, st1, r2(g1_0), r2(bt1_0), W2_0, r2(b2_0), 2 * D, D)
    hv1, t0, t1, t2, t3 = _p2(X2, st2, r2(g2_0), r2(bt2_0))
    shp = _sc_spmm(t0, t1, t2, t3, src2, dst2, z16)
    X3, st3 = _p3(shp, Se, hv1, deg, r2(eps1), W1_1, r2(b1_1))
    X4, st4 = _p1(X3, st3, r2(g1_1), r2(bt1_1), W2_1, r2(b2_1), 2 * D, D)
    return _p5(X4, st4, r2(g2_1), r2(bt2_1), Wp, r2(bp))


# speedup vs baseline: 2.2736x; 1.0386x over previous
import functools
import jax
import jax.numpy as jnp
from jax import lax
from jax.experimental import pallas as pl
from jax.experimental.pallas import tpu as pltpu
from jax.experimental.pallas import tpu_sc as plsc

N = 100000
E = 1600000
IN_DIM = 16
D = 50
NUM_TASK = 112
FN = jnp.float32

CHUNK = 128
GCHUNKS = 8
EGROUP = CHUNK * GCHUNKS
GROUPS_PER_TILE = 49
NGROUPS = 32 * GROUPS_PER_TILE
EPAD = NGROUPS * EGROUP
ROWS_PER_TILE = 6256
NPAD = 16 * ROWS_PER_TILE
CW = 16
NCH = 4

_mesh = plsc.VectorSubcoreMesh(core_axis_name="c", subcore_axis_name="s")
_sc_params = pltpu.CompilerParams(use_tc_tiling_on_sc=False)

_ACC_T = jax.ShapeDtypeStruct((2, NPAD, CW), jnp.float32)


def _sc_scatter_body(h0, h1, h2, h3, dst2, z16, o0, o1, o2, o3,
                     acc, ibuf, fbuf, sem):
    c, s = lax.axis_index("c"), lax.axis_index("s")
    row0 = s * ROWS_PER_TILE
    g0 = (c * 16 + s) * GROUPS_PER_TILE

    for tab, out in ((h0, o0), (h1, o1), (h2, o2), (h3, o3)):
        pltpu.sync_copy(z16.at[pl.ds(row0, ROWS_PER_TILE), :],
                        acc.at[pl.ds(row0, ROWS_PER_TILE), :])
        plsc.subcore_barrier()

        def body(k2, carry):
            g = g0 + k2
            pltpu.sync_copy(dst2.at[pl.ds(g * GCHUNKS, GCHUNKS), :], ibuf)
            pltpu.sync_copy(tab.at[pl.ds(g * EGROUP, EGROUP), :], fbuf)
            for j in range(GCHUNKS):
                pltpu.sync_copy(fbuf.at[pl.ds(j * CHUNK, CHUNK), :],
                                acc.at[ibuf.at[j]], add=True)
            return carry

        lax.fori_loop(0, GROUPS_PER_TILE, body, 0)
        plsc.subcore_barrier()
        pltpu.sync_copy(acc.at[pl.ds(row0, ROWS_PER_TILE), :],
                        out.at[c, pl.ds(row0, ROWS_PER_TILE), :])
        plsc.subcore_barrier()


_sc_scatter = functools.partial(
    pl.kernel,
    mesh=_mesh,
    out_type=[_ACC_T, _ACC_T, _ACC_T, _ACC_T],
    scratch_types=[
        pltpu.VMEM_SHARED((NPAD, CW), jnp.float32),
        pltpu.VMEM((GCHUNKS, CHUNK), jnp.int32),
        pltpu.VMEM((EGROUP, CW), jnp.float32),
        pltpu.SemaphoreType.DMA,
    ],
    compiler_params=_sc_params,
)(_sc_scatter_body)


def _sc_spmm_body(t0, t1, t2, t3, src2, dst2, z16, o0, o1, o2, o3,
                  acc, sibuf, dibuf, vbuf, sem):
    c, s = lax.axis_index("c"), lax.axis_index("s")
    row0 = s * ROWS_PER_TILE
    g0 = (c * 16 + s) * GROUPS_PER_TILE

    for tab, out in ((t0, o0), (t1, o1), (t2, o2), (t3, o3)):
        pltpu.sync_copy(z16.at[pl.ds(row0, ROWS_PER_TILE), :],
                        acc.at[pl.ds(row0, ROWS_PER_TILE), :])
        plsc.subcore_barrier()

        def body(k2, carry):
            g = g0 + k2
            pltpu.sync_copy(src2.at[pl.ds(g * GCHUNKS, GCHUNKS), :], sibuf)
            pltpu.sync_copy(dst2.at[pl.ds(g * GCHUNKS, GCHUNKS), :], dibuf)
            for j in range(GCHUNKS):
                pltpu.async_copy(tab.at[sibuf.at[j]], vbuf, sem).wait()
                pltpu.sync_copy(vbuf, acc.at[dibuf.at[j]], add=True)
            return carry

        lax.fori_loop(0, GROUPS_PER_TILE, body, 0)
        plsc.subcore_barrier()
        pltpu.sync_copy(acc.at[pl.ds(row0, ROWS_PER_TILE), :],
                        out.at[c, pl.ds(row0, ROWS_PER_TILE), :])
        plsc.subcore_barrier()


_sc_spmm = functools.partial(
    pl.kernel,
    mesh=_mesh,
    out_type=[_ACC_T, _ACC_T, _ACC_T, _ACC_T],
    scratch_types=[
        pltpu.VMEM_SHARED((NPAD, CW), jnp.float32),
        pltpu.VMEM((GCHUNKS, CHUNK), jnp.int32),
        pltpu.VMEM((GCHUNKS, CHUNK), jnp.int32),
        pltpu.VMEM((CHUNK, CW), jnp.float32),
        pltpu.SemaphoreType.DMA,
    ],
    compiler_params=_sc_params,
)(_sc_spmm_body)


BROWS = 4000
GRID = N // BROWS
EROWS = 8192
EGRID = EPAD // EROWS


def _row_spec(width, rows=BROWS):
    return pl.BlockSpec((rows, width), lambda i: (i, 0))


def _part_spec(width):
    return pl.BlockSpec((2, BROWS, width), lambda i: (0, i, 0))


def _full_spec(a, b):
    return pl.BlockSpec((a, b), lambda i: (0, 0))


def _he_body(ef, We, be, c0, c1, c2, c3):
    x = jnp.dot(ef[...], We[...], preferred_element_type=FN) + be[...]
    c0[...] = x[:, 0:16]
    c1[...] = x[:, 16:32]
    c2[...] = x[:, 32:48]
    c3[...] = jnp.concatenate(
        [x[:, 48:50], jnp.ones((EROWS, 1), FN), jnp.zeros((EROWS, 13), FN)],
        axis=1)


def _he(efp, We, be):
    return pl.pallas_call(
        _he_body,
        grid=(EGRID,),
        in_specs=[_row_spec(IN_DIM, EROWS), _full_spec(IN_DIM, D),
                  _full_spec(1, D)],
        out_specs=[_row_spec(CW, EROWS)] * NCH,
        out_shape=[jax.ShapeDtypeStruct((EPAD, CW), FN)] * NCH,
    )(efp, We, be)


def _p0_body(c0p, c1p, c2p, c3p, nemb, eps0, W1, b1,
             se_out, x1_out, st_out, deg_out):
    i = pl.program_id(0)
    s3 = c3p[0] + c3p[1]
    deg = s3[:, 2]
    deg_out[...] = deg[:, None]
    Se = jnp.concatenate(
        [c0p[0] + c0p[1], c1p[0] + c1p[1], c2p[0] + c2p[1], s3[:, 0:2]],
        axis=1)
    se_out[...] = Se
    maxdeg = jnp.maximum(deg, 1.0)[:, None]
    r = nemb[...]
    h = (1.0 + eps0[0, 0]) * r + (deg[:, None] * r + Se) / maxdeg
    x = jnp.dot(h, W1[...], preferred_element_type=FN) + b1[...]
    x1_out[...] = x
    st = jnp.concatenate([jnp.sum(x, axis=0, keepdims=True),
                          jnp.sum(x * x, axis=0, keepdims=True)], axis=0)

    @pl.when(i == 0)
    def _():
        st_out[...] = jnp.zeros_like(st_out)

    st_out[...] += st


def _p0(sep, nemb, eps0, W1, b1):
    return pl.pallas_call(
        _p0_body,
        grid=(GRID,),
        in_specs=[
            _part_spec(CW), _part_spec(CW), _part_spec(CW), _part_spec(CW),
            _full_spec(1, D), _full_spec(1, 1),
            _full_spec(D, 2 * D), _full_spec(1, 2 * D),
        ],
        out_specs=[
            _row_spec(D), _row_spec(2 * D),
            pl.BlockSpec((2, 2 * D), lambda i: (0, 0)), _row_spec(1),
        ],
        out_shape=[
            jax.ShapeDtypeStruct((N, D), FN),
            jax.ShapeDtypeStruct((N, 2 * D), FN),
            jax.ShapeDtypeStruct((2, 2 * D), FN),
            jax.ShapeDtypeStruct((N, 1), FN),
        ],
    )(sep[0], sep[1], sep[2], sep[3], nemb, eps0, W1, b1)


def _bn_from_stats(x, st, g, bt):
    mean = st[0:1, :] * (1.0 / N)
    var = st[1:2, :] * (1.0 / N) - mean * mean
    inv = lax.rsqrt(var + 1e-5)
    return (x - mean) * inv * g + bt


def _p1_body(x_in, st_in, g, bt, W2, b2, y_out, st_out):
    i = pl.program_id(0)
    xn = _bn_from_stats(x_in[...], st_in[...], g[...], bt[...])
    xn = jnp.maximum(xn, 0.0)
    y = jnp.dot(xn, W2[...], preferred_element_type=FN) + b2[...]
    y_out[...] = y
    st = jnp.concatenate([jnp.sum(y, axis=0, keepdims=True),
                          jnp.sum(y * y, axis=0, keepdims=True)], axis=0)

    @pl.when(i == 0)
    def _():
        st_out[...] = jnp.zeros_like(st_out)

    st_out[...] += st


def _p1(x, st, g, bt, W2, b2, din, dout):
    return pl.pallas_call(
        _p1_body,
        grid=(GRID,),
        in_specs=[
            _row_spec(din), pl.BlockSpec((2, din), lambda i: (0, 0)),
            _full_spec(1, din), _full_spec(1, din),
            _full_spec(din, dout), _full_spec(1, dout),
        ],
        out_specs=[_row_spec(dout), pl.BlockSpec((2, dout), lambda i: (0, 0))],
        out_shape=[
            jax.ShapeDtypeStruct((N, dout), FN),
            jax.ShapeDtypeStruct((2, dout), FN),
        ],
    )(x, st, g, bt, W2, b2)


def _p2_body(x_in, st_in, g, bt, hv_out, t0, t1, t2, t3):
    h = _bn_from_stats(x_in[...], st_in[...], g[...], bt[...])
    h = jnp.maximum(h, 0.0)
    hv_out[...] = h
    t0[...] = h[:, 0:16]
    t1[...] = h[:, 16:32]
    t2[...] = h[:, 32:48]
    t3[...] = jnp.concatenate([h[:, 48:50], jnp.zeros((BROWS, 14), FN)],
                              axis=1)


def _p2(x, st, g, bt):
    return pl.pallas_call(
        _p2_body,
        grid=(GRID,),
        in_specs=[
            _row_spec(D), pl.BlockSpec((2, D), lambda i: (0, 0)),
            _full_spec(1, D), _full_spec(1, D),
        ],
        out_specs=[_row_spec(D)] + [_row_spec(CW)] * NCH,
        out_shape=[jax.ShapeDtypeStruct((N, D), FN)]
        + [jax.ShapeDtypeStruct((N, CW), FN)] * NCH,
    )(x, st, g, bt)


def _p3_body(q0, q1, q2, q3, se_in, hv_in, dg, eps1, W1, b1, x_out, st_out):
    i = pl.program_id(0)
    deg = dg[:, 0]
    maxdeg = jnp.maximum(deg, 1.0)[:, None]
    Sh = jnp.concatenate(
        [q0[0] + q0[1], q1[0] + q1[1], q2[0] + q2[1],
         (q3[0] + q3[1])[:, 0:2]], axis=1)
    h = (1.0 + eps1[0, 0]) * hv_in[...] + (Sh + se_in[...]) / maxdeg
    x = jnp.dot(h, W1[...], preferred_element_type=FN) + b1[...]
    x_out[...] = x
    st = jnp.concatenate([jnp.sum(x, axis=0, keepdims=True),
                          jnp.sum(x * x, axis=0, keepdims=True)], axis=0)

    @pl.when(i == 0)
    def _():
        st_out[...] = jnp.zeros_like(st_out)

    st_out[...] += st


def _p3(shp, Se, hv1, deg, eps1, W1, b1):
    return pl.pallas_call(
        _p3_body,
        grid=(GRID,),
        in_specs=[
            _part_spec(CW), _part_spec(CW), _part_spec(CW), _part_spec(CW),
            _row_spec(D), _row_spec(D), _row_spec(1),
            _full_spec(1, 1), _full_spec(D, 2 * D), _full_spec(1, 2 * D),
        ],
        out_specs=[_row_spec(2 * D), pl.BlockSpec((2, 2 * D), lambda i: (0, 0))],
        out_shape=[
            jax.ShapeDtypeStruct((N, 2 * D), FN),
            jax.ShapeDtypeStruct((2, 2 * D), FN),
        ],
    )(shp[0], shp[1], shp[2], shp[3], Se, hv1, deg, eps1, W1, b1)


def _p5_body(x_in, st_in, g, bt, Wp, bp, o_out):
    h = _bn_from_stats(x_in[...], st_in[...], g[...], bt[...])
    h = jnp.maximum(h, 0.0)
    o_out[...] = jnp.dot(h, Wp[...], preferred_element_type=FN) + bp[...]


def _p5(x, st, g, bt, Wp, bp):
    return pl.pallas_call(
        _p5_body,
        grid=(GRID,),
        in_specs=[
            _row_spec(D), pl.BlockSpec((2, D), lambda i: (0, 0)),
            _full_spec(1, D), _full_spec(1, D),
            _full_spec(D, NUM_TASK), _full_spec(1, NUM_TASK),
        ],
        out_specs=[_row_spec(NUM_TASK)],
        out_shape=[jax.ShapeDtypeStruct((N, NUM_TASK), FN)],
    )(x, st, g, bt, Wp, bp)[0]


def kernel(edge_index, edge_feats, node_emb, We, be, eps0, W1_0, b1_0, g1_0,
           bt1_0, W2_0, b2_0, g2_0, bt2_0, eps1, W1_1, b1_1, g1_1, bt1_1,
           W2_1, b2_1, g2_1, bt2_1, Wp, bp):
    npad_e = EPAD - E
    src2 = jnp.concatenate(
        [edge_index[0], jnp.zeros((npad_e,), jnp.int32)]).reshape(-1, CHUNK)
    dst2 = jnp.concatenate(
        [edge_index[1], jnp.full((npad_e,), N, jnp.int32)]).reshape(-1, CHUNK)
    efp = jnp.concatenate([edge_feats, jnp.zeros((npad_e, IN_DIM), FN)], 0)
    z16 = jnp.zeros((NPAD, CW), FN)

    r2 = lambda v: v.reshape(1, -1)

    hec = _he(efp, We, r2(be))
    sep = _sc_scatter(hec[0], hec[1], hec[2], hec[3], dst2, z16)
    Se, X1, st1, deg = _p0(sep, node_emb, r2(eps0), W1_0, r2(b1_0))
    X2, st2 = _p1(X1, st1, r2(g1_0), r2(bt1_0), W2_0, r2(b2_0), 2 * D, D)
    hv1, t0, t1, t2, t3 = _p2(X2, st2, r2(g2_0), r2(bt2_0))
    shp = _sc_spmm(t0, t1, t2, t3, src2, dst2, z16)
    X3, st3 = _p3(shp, Se, hv1, deg, r2(eps1), W1_1, r2(b1_1))
    X4, st4 = _p1(X3, st3, r2(g1_1), r2(bt1_1), W2_1, r2(b2_1), 2 * D, D)
    return _p5(X4, st4, r2(g2_1), r2(bt2_1), Wp, r2(bp))
